# Initial kernel scaffold; baseline (speedup 1.0000x reference)
import jax
import jax.numpy as jnp
from jax.experimental import pallas as pl


def kernel(x, edge_index, batch, params):
    raise NotImplementedError("write your pallas kernel here")


# speedup vs baseline: 4.1174x; 4.1174x over previous
import functools

import jax
import jax.numpy as jnp
from jax import lax
from jax.experimental import pallas as pl
from jax.experimental.pallas import tpu as pltpu
from jax.experimental.pallas import tpu_sc as plsc

_G = 64
_NW = 32
_CH = 128


def _make_sc_aggregate(n, d, nchunks, npad):
    stripe = npad // 16
    mesh = plsc.VectorSubcoreMesh(core_axis_name="c", subcore_axis_name="s")

    @functools.partial(
        pl.kernel,
        mesh=mesh,
        out_type=jax.ShapeDtypeStruct((2, npad, d), jnp.float32),
        scratch_types=[
            pltpu.VMEM((nchunks, _CH), jnp.int32),
            pltpu.VMEM((nchunks, _CH), jnp.int32),
            pltpu.VMEM((_CH, d), jnp.float32),
            pltpu.VMEM_SHARED((npad, d), jnp.float32),
            pltpu.SemaphoreType.DMA,
        ],
    )
    def sc_agg(z_hbm, src_hbm, dst_hbm, zero_hbm, out_hbm,
               src_v, dst_v, row_v, agg_sh, sem):
        cid = lax.axis_index("c")
        sid = lax.axis_index("s")
        wid = cid * 16 + sid
        pltpu.sync_copy(src_hbm.at[wid], src_v)
        pltpu.sync_copy(dst_hbm.at[wid], dst_v)
        r0 = sid * stripe
        pltpu.sync_copy(zero_hbm.at[pl.ds(r0, stripe)],
                        agg_sh.at[pl.ds(r0, stripe)])
        plsc.subcore_barrier()

        def body(t, carry):
            pltpu.async_copy(z_hbm.at[src_v.at[t]], row_v, sem).wait()
            pltpu.sync_copy(row_v, agg_sh.at[dst_v.at[t]], add=True)
            return carry

        lax.fori_loop(0, nchunks, body, 0)
        plsc.subcore_barrier()
        pltpu.sync_copy(agg_sh.at[pl.ds(r0, stripe)],
                        out_hbm.at[cid, pl.ds(r0, stripe)])

    return sc_agg


def _make_tc_layer(n, d, h, g):
    def body(z_ref, agg_ref, w1_ref, b1_ref, w2_ref, b2_ref,
             gam_ref, bet_ref, bat_ref, zo_ref, go_ref):
        a = agg_ref[0] + agg_ref[1]
        x0 = z_ref[...] + a[0:n]
        x1 = jnp.dot(x0, w1_ref[...], preferred_element_type=jnp.float32)
        x1 = jnp.maximum(x1 + b1_ref[...][None, :], 0.0)
        x2 = jnp.dot(x1, w2_ref[...], preferred_element_type=jnp.float32)
        x2 = jnp.maximum(x2 + b2_ref[...][None, :], 0.0)
        mean = jnp.mean(x2, axis=0)
        c = x2 - mean[None, :]
        var = jnp.mean(c * c, axis=0)
        scale = lax.rsqrt(var + 1e-5) * gam_ref[...]
        hn = c * scale[None, :] + bet_ref[...][None, :]
        zo_ref[...] = hn
        onehot = (bat_ref[...] == lax.broadcasted_iota(
            jnp.int32, (g, n), 0)).astype(jnp.float32)
        go_ref[...] = jnp.dot(onehot, hn, preferred_element_type=jnp.float32)

    return pl.pallas_call(
        body,
        out_shape=(
            jax.ShapeDtypeStruct((n, h), jnp.float32),
            jax.ShapeDtypeStruct((g, h), jnp.float32),
        ),
    )


def kernel(x, edge_index, batch, params):
    n, d = x.shape
    e = edge_index.shape[1]
    hdim = params[0][0].shape[1]
    g = _G

    per_w = -(-e // _NW)
    nchunks = -(-per_w // _CH)
    epad = _NW * nchunks * _CH
    npad = -(-(n + 1) // 128) * 128

    src = edge_index[0]
    dst = edge_index[1]
    pad = epad - e
    src_p = jnp.concatenate(
        [src, jnp.zeros((pad,), jnp.int32)]).reshape(_NW, nchunks, _CH)
    dst_p = jnp.concatenate(
        [dst, jnp.full((pad,), n, jnp.int32)]).reshape(_NW, nchunks, _CH)
    zero_hbm = jnp.zeros((npad, d), jnp.float32)
    bat2 = batch[None, :].astype(jnp.int32)

    sc_agg = _make_sc_aggregate(n, d, nchunks, npad)

    z = x
    zs = []
    gs = []
    for li, (w1, b1, w2, b2, gamma, beta) in enumerate(params):
        agg2 = sc_agg(z, src_p, dst_p, zero_hbm)
        tc = _make_tc_layer(n, d if li == 0 else hdim, hdim, g)
        z, gp = tc(z, agg2, w1, b1, w2, b2, gamma, beta, bat2)
        zs.append(z)
        gs.append(gp)
    return jnp.concatenate(zs, axis=1), jnp.concatenate(gs, axis=1)
